# baseline (speedup 1.0000x reference)
import functools

import jax
import jax.numpy as jnp
from jax import lax
from jax.experimental import pallas as pl
from jax.experimental.pallas import tpu as pltpu
from jax.experimental.pallas import tpu_sc as plsc

NC = 2
NS = 16
L = 16
NW = NC * NS
B = 16384
D = 128
RPW = B // NW
CH = 128
NCHUNK = RPW // CH
G = CH // L


def _mf_body(uid_hbm, iid_hbm, uemb_hbm, iemb_hbm, ubias_hbm, ibias_hbm,
             gbias_hbm, out_hbm,
             uid_v, iid_v, ubuf, ibuf, ub_v, ib_v, out_v, pbuf, gb_v, sem):
    wid = lax.axis_index("s") * NC + lax.axis_index("c")
    base_chunk = wid * NCHUNK

    pltpu.sync_copy(uid_hbm.at[pl.ds(base_chunk, NCHUNK)], uid_v)
    pltpu.sync_copy(iid_hbm.at[pl.ds(base_chunk, NCHUNK)], iid_v)
    pltpu.sync_copy(gbias_hbm, gb_v)
    gb = gb_v[...]

    for c in range(NCHUNK):
        cp1 = pltpu.async_copy(ubias_hbm.at[uid_v.at[c]], ub_v.at[c], sem)
        cp2 = pltpu.async_copy(ibias_hbm.at[iid_v.at[c]], ib_v.at[c], sem)
        cp1.wait()
        cp2.wait()

    iota = lax.iota(jnp.int32, 16)

    for c in range(NCHUNK):
        cp1 = pltpu.async_copy(uemb_hbm.at[uid_v.at[c]], ubuf, sem)
        cp2 = pltpu.async_copy(iemb_hbm.at[iid_v.at[c]], ibuf, sem)
        cp1.wait()
        cp2.wait()

        def group_body(g, _, c=c):
            for r in range(L):
                row = g * L + r
                acc = ubuf[row, pl.ds(0, L)] * ibuf[row, pl.ds(0, L)]
                for k in range(1, D // L):
                    acc = acc + ubuf[row, pl.ds(k * L, L)] * ibuf[row, pl.ds(k * L, L)]
                pbuf[pl.ds(r * L, L)] = acc
            row_base = iota * L
            tot = plsc.load_gather(pbuf, [row_base])
            for l in range(1, L):
                tot = tot + plsc.load_gather(pbuf, [row_base + l])
            tot = tot + ub_v[c, pl.ds(g * L, L)] + ib_v[c, pl.ds(g * L, L)] + gb
            out_v[pl.ds(c * CH + g * L, L)] = tot
            return 0

        lax.fori_loop(0, G, group_body, 0)

    pltpu.sync_copy(out_v, out_hbm.at[pl.ds(wid * RPW, RPW)])


_mf = pl.kernel(
    _mf_body,
    out_type=jax.ShapeDtypeStruct((B,), jnp.float32),
    mesh=plsc.VectorSubcoreMesh(
        core_axis_name="c", subcore_axis_name="s",
        num_cores=NC, num_subcores=NS),
    scratch_types=[
        pltpu.VMEM((NCHUNK, CH), jnp.int32),
        pltpu.VMEM((NCHUNK, CH), jnp.int32),
        pltpu.VMEM((CH, D), jnp.float32),
        pltpu.VMEM((CH, D), jnp.float32),
        pltpu.VMEM((NCHUNK, CH), jnp.float32),
        pltpu.VMEM((NCHUNK, CH), jnp.float32),
        pltpu.VMEM((RPW,), jnp.float32),
        pltpu.VMEM((L * L,), jnp.float32),
        pltpu.VMEM((L,), jnp.float32),
        pltpu.SemaphoreType.DMA,
    ],
    compiler_params=pltpu.CompilerParams(needs_layout_passes=False),
)


@jax.jit
def kernel(user_id, item_id, user_embedding, item_embedding, user_bias,
           item_bias, global_bias):
    uid2 = user_id.astype(jnp.int32).reshape(B // CH, CH)
    iid2 = item_id.astype(jnp.int32).reshape(B // CH, CH)
    ub = user_bias.reshape(-1)
    ib = item_bias.reshape(-1)
    gb16 = jnp.broadcast_to(global_bias.reshape(()), (L,))
    return _mf(uid2, iid2, user_embedding, item_embedding, ub, ib, gb16)


# speedup vs baseline: 1.3144x; 1.3144x over previous
import functools

import jax
import jax.numpy as jnp
from jax import lax
from jax.experimental import pallas as pl
from jax.experimental.pallas import tpu as pltpu
from jax.experimental.pallas import tpu_sc as plsc

NC = 2
NS = 16
L = 16
NW = NC * NS
B = 16384
D = 128
RPW = B // NW
CH = 128
NCHUNK = RPW // CH
G = CH // L

_MESH = plsc.VectorSubcoreMesh(
    core_axis_name="c", subcore_axis_name="s",
    num_cores=NC, num_subcores=NS)


def _dot_body(uid_hbm, iid_hbm, uemb_hbm, iemb_hbm, out_hbm,
              uid_v, iid_v, ubuf0, ibuf0, ubuf1, ibuf1, out_v, pbuf,
              sem0, sem1):
    wid = lax.axis_index("s") * NC + lax.axis_index("c")
    base_chunk = wid * NCHUNK

    pltpu.sync_copy(uid_hbm.at[pl.ds(base_chunk, NCHUNK)], uid_v)
    pltpu.sync_copy(iid_hbm.at[pl.ds(base_chunk, NCHUNK)], iid_v)

    ubufs = (ubuf0, ubuf1)
    ibufs = (ibuf0, ibuf1)
    sems = (sem0, sem1)
    iota = lax.iota(jnp.int32, L)

    pltpu.async_copy(uemb_hbm.at[uid_v.at[0]], ubufs[0], sems[0])
    pltpu.async_copy(iemb_hbm.at[iid_v.at[0]], ibufs[0], sems[0])

    for c in range(NCHUNK):
        par = c % 2
        if c + 1 < NCHUNK:
            nxt = (c + 1) % 2
            pltpu.async_copy(uemb_hbm.at[uid_v.at[c + 1]], ubufs[nxt], sems[nxt])
            pltpu.async_copy(iemb_hbm.at[iid_v.at[c + 1]], ibufs[nxt], sems[nxt])
        pltpu.make_async_copy(uemb_hbm.at[uid_v.at[c]], ubufs[par], sems[par]).wait()
        pltpu.make_async_copy(iemb_hbm.at[iid_v.at[c]], ibufs[par], sems[par]).wait()
        ubuf = ubufs[par]
        ibuf = ibufs[par]

        def group_body(g, _, ubuf=ubuf, ibuf=ibuf, c=c):
            for r in range(L):
                row = g * L + r
                acc = ubuf[row, pl.ds(0, L)] * ibuf[row, pl.ds(0, L)]
                for k in range(1, D // L):
                    acc = acc + ubuf[row, pl.ds(k * L, L)] * ibuf[row, pl.ds(k * L, L)]
                pbuf[pl.ds(r * L, L)] = acc
            row_base = iota * L
            tot = plsc.load_gather(pbuf, [row_base])
            for l in range(1, L):
                tot = tot + plsc.load_gather(pbuf, [row_base + l])
            out_v[pl.ds(c * CH + g * L, L)] = tot
            return 0

        lax.fori_loop(0, G, group_body, 0)

    pltpu.sync_copy(out_v, out_hbm.at[pl.ds(wid * RPW, RPW)])


_mf_dot = pl.kernel(
    _dot_body,
    out_type=jax.ShapeDtypeStruct((B,), jnp.float32),
    mesh=_MESH,
    scratch_types=[
        pltpu.VMEM((NCHUNK, CH), jnp.int32),
        pltpu.VMEM((NCHUNK, CH), jnp.int32),
        pltpu.VMEM((CH, D), jnp.float32),
        pltpu.VMEM((CH, D), jnp.float32),
        pltpu.VMEM((CH, D), jnp.float32),
        pltpu.VMEM((CH, D), jnp.float32),
        pltpu.VMEM((RPW,), jnp.float32),
        pltpu.VMEM((L * L,), jnp.float32),
        pltpu.SemaphoreType.DMA,
        pltpu.SemaphoreType.DMA,
    ],
    compiler_params=pltpu.CompilerParams(needs_layout_passes=False),
)


def _bias_body(uid_hbm, iid_hbm, ub_hbm, ib_hbm, gb_hbm, dots_hbm, out_hbm,
               uid_v, iid_v, ub_v, ib_v, dots_v, out_v, gb_v, sem):
    wid = lax.axis_index("s") * NC + lax.axis_index("c")
    base_chunk = wid * NCHUNK

    pltpu.sync_copy(uid_hbm.at[pl.ds(base_chunk, NCHUNK)], uid_v)
    pltpu.sync_copy(iid_hbm.at[pl.ds(base_chunk, NCHUNK)], iid_v)
    pltpu.sync_copy(gb_hbm, gb_v)

    cps = []
    for c in range(NCHUNK):
        cps.append(pltpu.async_copy(
            ub_hbm.at[uid_v.at[c]], ub_v.at[pl.ds(c * CH, CH)], sem))
        cps.append(pltpu.async_copy(
            ib_hbm.at[iid_v.at[c]], ib_v.at[pl.ds(c * CH, CH)], sem))
    pltpu.sync_copy(dots_hbm.at[pl.ds(wid * RPW, RPW)], dots_v)
    for cp in cps:
        cp.wait()

    gb = gb_v[...]

    def group_body(g, _):
        off = g * L
        tot = (dots_v[pl.ds(off, L)] + gb
               + ub_v[pl.ds(off, L)] + ib_v[pl.ds(off, L)])
        out_v[pl.ds(off, L)] = tot
        return 0

    lax.fori_loop(0, RPW // L, group_body, 0)

    pltpu.sync_copy(out_v, out_hbm.at[pl.ds(wid * RPW, RPW)])


_mf_bias = pl.kernel(
    _bias_body,
    out_type=jax.ShapeDtypeStruct((B,), jnp.float32),
    mesh=_MESH,
    scratch_types=[
        pltpu.VMEM((NCHUNK, CH), jnp.int32),
        pltpu.VMEM((NCHUNK, CH), jnp.int32),
        pltpu.VMEM((RPW,), jnp.float32),
        pltpu.VMEM((RPW,), jnp.float32),
        pltpu.VMEM((RPW,), jnp.float32),
        pltpu.VMEM((RPW,), jnp.float32),
        pltpu.VMEM((L,), jnp.float32),
        pltpu.SemaphoreType.DMA,
    ],
    compiler_params=pltpu.CompilerParams(needs_layout_passes=False),
)


@jax.jit
def kernel(user_id, item_id, user_embedding, item_embedding, user_bias,
           item_bias, global_bias):
    uid2 = user_id.astype(jnp.int32).reshape(B // CH, CH)
    iid2 = item_id.astype(jnp.int32).reshape(B // CH, CH)
    ub1 = user_bias.reshape(-1)
    ib1 = item_bias.reshape(-1)
    gb16 = jnp.broadcast_to(global_bias.reshape(()), (L,))
    dots = _mf_dot(uid2, iid2, user_embedding, item_embedding)
    return _mf_bias(uid2, iid2, ub1, ib1, gb16, dots)


# speedup vs baseline: 1.4465x; 1.1005x over previous
import jax
import jax.numpy as jnp
from jax import lax
from jax.experimental import pallas as pl
from jax.experimental.pallas import tpu as pltpu
from jax.experimental.pallas import tpu_sc as plsc

NC = 2
NS = 16
L = 16
NW = NC * NS
B = 16384
D = 128
RPW = B // NW
CH = 128
NCHUNK = RPW // CH
G = CH // L

_MESH = plsc.VectorSubcoreMesh(
    core_axis_name="c", subcore_axis_name="s",
    num_cores=NC, num_subcores=NS)


def _dot_body(uid_hbm, iid_hbm, uemb_hbm, iemb_hbm, out_hbm,
              uid_v, iid_v, ubuf0, ibuf0, ubuf1, ibuf1, out_v, pbuf,
              sem0, sem1):
    wid = lax.axis_index("s") * NC + lax.axis_index("c")
    base_chunk = wid * NCHUNK

    pltpu.sync_copy(uid_hbm.at[pl.ds(base_chunk, NCHUNK)], uid_v)
    pltpu.sync_copy(iid_hbm.at[pl.ds(base_chunk, NCHUNK)], iid_v)

    ubufs = (ubuf0, ubuf1)
    ibufs = (ibuf0, ibuf1)
    sems = (sem0, sem1)
    iota = lax.iota(jnp.int32, L)

    pltpu.async_copy(uemb_hbm.at[uid_v.at[0]], ubufs[0], sems[0])
    pltpu.async_copy(iemb_hbm.at[iid_v.at[0]], ibufs[0], sems[0])

    for c in range(NCHUNK):
        par = c % 2
        if c + 1 < NCHUNK:
            nxt = (c + 1) % 2
            pltpu.async_copy(uemb_hbm.at[uid_v.at[c + 1]], ubufs[nxt], sems[nxt])
            pltpu.async_copy(iemb_hbm.at[iid_v.at[c + 1]], ibufs[nxt], sems[nxt])
        pltpu.make_async_copy(uemb_hbm.at[uid_v.at[c]], ubufs[par], sems[par]).wait()
        pltpu.make_async_copy(iemb_hbm.at[iid_v.at[c]], ibufs[par], sems[par]).wait()
        ubuf = ubufs[par]
        ibuf = ibufs[par]

        def group_body(g, _, ubuf=ubuf, ibuf=ibuf, c=c):
            for r in range(L):
                row = g * L + r
                acc = ubuf[row, pl.ds(0, L)] * ibuf[row, pl.ds(0, L)]
                for k in range(1, D // L):
                    acc = acc + ubuf[row, pl.ds(k * L, L)] * ibuf[row, pl.ds(k * L, L)]
                pbuf[pl.ds(r * L, L)] = acc
            row_base = iota * L
            tot = plsc.load_gather(pbuf, [row_base])
            for l in range(1, L):
                tot = tot + plsc.load_gather(pbuf, [row_base + l])
            out_v[pl.ds(c * CH + g * L, L)] = tot
            return 0

        lax.fori_loop(0, G, group_body, 0)

    pltpu.sync_copy(out_v, out_hbm.at[pl.ds(wid * RPW, RPW)])


_mf_dot = pl.kernel(
    _dot_body,
    out_type=jax.ShapeDtypeStruct((B,), jnp.float32),
    mesh=_MESH,
    scratch_types=[
        pltpu.VMEM((NCHUNK, CH), jnp.int32),
        pltpu.VMEM((NCHUNK, CH), jnp.int32),
        pltpu.VMEM((CH, D), jnp.float32),
        pltpu.VMEM((CH, D), jnp.float32),
        pltpu.VMEM((CH, D), jnp.float32),
        pltpu.VMEM((CH, D), jnp.float32),
        pltpu.VMEM((RPW,), jnp.float32),
        pltpu.VMEM((L * L,), jnp.float32),
        pltpu.SemaphoreType.DMA,
        pltpu.SemaphoreType.DMA,
    ],
    compiler_params=pltpu.CompilerParams(needs_layout_passes=False),
)


def _bias_body(uid_hbm, iid_hbm, ub_hbm, ib_hbm, dots_hbm, out_hbm,
               uid_v, iid_v, ub_v, ib_v, dots_v, out_v, sem, semi):
    wid = lax.axis_index("s") * NC + lax.axis_index("c")
    base_chunk = wid * NCHUNK

    cpu = pltpu.async_copy(uid_hbm.at[pl.ds(base_chunk, NCHUNK)], uid_v, semi)
    cpi = pltpu.async_copy(iid_hbm.at[pl.ds(base_chunk, NCHUNK)], iid_v, semi)
    cpd = pltpu.async_copy(dots_hbm.at[pl.ds(wid * RPW, RPW)], dots_v, semi)
    cpu.wait()
    cpi.wait()

    cps = []
    for c in range(NCHUNK):
        cps.append(pltpu.async_copy(
            ub_hbm.at[uid_v.at[c]], ub_v.at[pl.ds(c * CH, CH)], sem))
        cps.append(pltpu.async_copy(
            ib_hbm.at[iid_v.at[c]], ib_v.at[pl.ds(c * CH, CH)], sem))
    cpd.wait()
    for cp in cps:
        cp.wait()

    def group_body(g, _):
        off = g * L
        tot = (dots_v[pl.ds(off, L)]
               + ub_v[pl.ds(off, L)] + ib_v[pl.ds(off, L)])
        out_v[pl.ds(off, L)] = tot
        return 0

    lax.fori_loop(0, RPW // L, group_body, 0)

    pltpu.sync_copy(out_v, out_hbm.at[pl.ds(wid * RPW, RPW)])


_mf_bias = pl.kernel(
    _bias_body,
    out_type=jax.ShapeDtypeStruct((B,), jnp.float32),
    mesh=_MESH,
    scratch_types=[
        pltpu.VMEM((NCHUNK, CH), jnp.int32),
        pltpu.VMEM((NCHUNK, CH), jnp.int32),
        pltpu.VMEM((RPW,), jnp.float32),
        pltpu.VMEM((RPW,), jnp.float32),
        pltpu.VMEM((RPW,), jnp.float32),
        pltpu.VMEM((RPW,), jnp.float32),
        pltpu.SemaphoreType.DMA,
        pltpu.SemaphoreType.DMA,
    ],
    compiler_params=pltpu.CompilerParams(needs_layout_passes=False),
)


@jax.jit
def kernel(user_id, item_id, user_embedding, item_embedding, user_bias,
           item_bias, global_bias):
    uid2 = user_id.astype(jnp.int32).reshape(B // CH, CH)
    iid2 = item_id.astype(jnp.int32).reshape(B // CH, CH)
    ub1 = user_bias.reshape(-1) + global_bias
    ib1 = item_bias.reshape(-1)
    dots = _mf_dot(uid2, iid2, user_embedding, item_embedding)
    return _mf_bias(uid2, iid2, ub1, ib1, dots)
